# baseline (speedup 1.0000x reference)
import dataclasses
import functools

import jax
import jax.numpy as jnp
from jax.experimental import pallas as pl
from jax.experimental.pallas import tpu as pltpu
from jax.experimental.pallas import tpu_sc as plsc

SD = 64
T = 4096
BLOOM = 16
NB = 16
N0 = T * BLOOM
HORIZON = 8
_INTERP = False

@functools.cache
def _sc_mesh():
    return plsc.VectorSubcoreMesh(core_axis_name="c", subcore_axis_name="s")


@functools.cache
def _sc_params():
    cp = pltpu.CompilerParams(use_tc_tiling_on_sc=False)
    if "needs_layout_passes" in pltpu.CompilerParams.__dataclass_fields__:
        cp = dataclasses.replace(cp, needs_layout_passes=False)
    return cp


def _preamble_body(s_ref, adv_ref, wm1, wm2, wa1, wa2, wp1, wp2, whs, whc,
                   strat_o, strat_a_o, strat_p_o, pi_o, psw_o, h0_o):
    s = s_ref[...]
    adv = adv_ref[...]
    strat = jnp.tanh(adv @ wm1[...] + s @ wm2[...])
    ps = jnp.tanh(strat @ wa1[...] + s @ wa2[...])
    pi = jnp.tanh(strat @ wp1[...] + ps @ wp2[...])
    strat_o[...] = strat
    strat_a_o[...] = strat @ wa1[...]
    strat_p_o[...] = strat @ wp1[...]
    pi_o[...] = pi
    psw_o[...] = ps @ whs[...]
    h0_o[...] = s @ whc[...]


def _round0_body(pi_ref, psw_ref, strat_c, whc, wha, wv, h0_ref, nz_ref,
                 cns_o, cv_o):
    ca = jnp.tanh(pi_ref[...] + nz_ref[...])
    cns = jnp.tanh(psw_ref[...] + ca @ wha[...])
    cns_o[...] = cns
    v = cns @ whc[...] - h0_ref[...]
    vp = (jnp.tanh(cns @ wv[...]) @ strat_c[...]) * ((HORIZON - 1.0) / HORIZON)
    cv_o[...] = v + vp


def _thresh_body(cv_ref, keys_o, pref_o, meta_o):
    v = cv_ref[...]
    s = jax.lax.bitcast_convert_type(v, jnp.int32)
    key = s ^ ((s >> 31) & jnp.int32(0x7FFFFFFF))
    keys_o[...] = key
    msb = jnp.int32(-2147483648)

    def bit(i, pu):
        cand_u = pu | (jnp.int32(1) << (31 - i))
        cand_s = cand_u ^ msb
        cnt = jnp.sum((key >= cand_s).astype(jnp.int32))
        return jnp.where(cnt >= T, cand_u, pu)

    pu = jax.lax.fori_loop(0, 32, bit, jnp.int32(0))
    t = pu ^ msb
    gt = key > t
    eq = key == t
    rem = T - jnp.sum(gt.astype(jnp.int32))
    ii = (jax.lax.broadcasted_iota(jnp.int32, (512, 128), 0) * 128
          + jax.lax.broadcasted_iota(jnp.int32, (512, 128), 1))

    def bit2(i, p0):
        cand = p0 | (jnp.int32(1) << (15 - i))
        c = jnp.sum((eq & (ii < cand)).astype(jnp.int32))
        return jnp.where(c < rem, cand, p0)

    p0 = jax.lax.fori_loop(0, 16, bit2, jnp.int32(0))
    mask = gt | (eq & (ii <= p0))
    mf = mask.astype(jnp.float32)
    sel16 = (jax.lax.broadcasted_iota(jnp.int32, (128, 8), 0) // 16
             == jax.lax.broadcasted_iota(jnp.int32, (128, 8), 1))
    cc = jnp.dot(mf, sel16.astype(jnp.float32))
    upper = (jax.lax.broadcasted_iota(jnp.int32, (8, 8), 0)
             < jax.lax.broadcasted_iota(jnp.int32, (8, 8), 1))
    rowpref = jnp.dot(cc, upper.astype(jnp.float32))
    rt = jnp.dot(cc, jnp.ones((8, 1), jnp.float32))
    lower = (jax.lax.broadcasted_iota(jnp.int32, (512, 512), 1)
             < jax.lax.broadcasted_iota(jnp.int32, (512, 512), 0))
    rtp = jnp.dot(lower.astype(jnp.float32), rt)
    pref_o[...] = (rowpref + rtp).astype(jnp.int32)
    meta_o[0] = t
    meta_o[1] = p0
    for q in range(2, 16):
        meta_o[q] = jnp.int32(0)


def _sc_select(first):
    def body(keys_hbm, meta_hbm, pref_hbm, anc_hbm, cns_hbm, cst_o, anc_o,
             kv, pv, mv, av, sb, ab, sp, ap, accs, acca, rows, spad, apad):
        cid = jax.lax.axis_index("c")
        sid = jax.lax.axis_index("s")

        @pl.when(cid == 0)
        def _():
            base = sid * 4096
            pltpu.sync_copy(keys_hbm.at[pl.ds(base, 4096)], kv)
            pltpu.sync_copy(pref_hbm.at[pl.ds(sid * 256, 256)],
                            pv.at[pl.ds(0, 256)])
            pltpu.sync_copy(meta_hbm, mv)
            if not first:
                pltpu.sync_copy(anc_hbm, av)
            mvv = mv[...]
            t = mvv[0]
            p0 = mvv[1]
            lane = jax.lax.iota(jnp.int32, 16)
            zero = lane - lane

            @pl.loop(0, 260)
            def _(c):
                sb[pl.ds(c * 16, 16)] = zero
                ab[pl.ds(c * 16, 16)] = zero

            @pl.loop(0, 256)
            def _(c):
                off = c * 16
                k16 = kv[pl.ds(off, 16)]
                gidx = lane + (base + off)
                m = (k16 > t) | ((k16 == t) & (gidx <= p0))
                mi = m.astype(jnp.int32)
                exc = plsc.cumsum(mi) - mi
                pvc = pv[pl.ds(c, 16)][0]
                dst = jnp.where(m, pvc + exc, T + lane)
                plsc.store_scatter(sb, [dst], gidx, mask=m)
                if first:
                    av_ = gidx
                else:
                    av_ = plsc.load_gather(av, [gidx & (T - 1)])
                plsc.store_scatter(ab, [dst], av_, mask=m)

            pltpu.sync_copy(sb, spad.at[sid])
            pltpu.sync_copy(ab, apad.at[sid])
            plsc.subcore_barrier()

            off2 = sid * 256
            pltpu.sync_copy(spad.at[:, pl.ds(off2, 256)], sp)
            pltpu.sync_copy(apad.at[:, pl.ds(off2, 256)], ap)

            @pl.loop(0, 16)
            def _(c):
                o16 = c * 16
                s = sp[0, pl.ds(o16, 16)]
                a = ap[0, pl.ds(o16, 16)]
                for r in range(1, 16):
                    s = s + sp[r, pl.ds(o16, 16)]
                    a = a + ap[r, pl.ds(o16, 16)]
                accs[pl.ds(o16, 16)] = s
                acca[pl.ds(o16, 16)] = a

            pltpu.sync_copy(cns_hbm.at[accs.at[pl.ds(0, 128)]], rows)
            pltpu.sync_copy(rows, cst_o.at[pl.ds(off2, 128)])
            pltpu.sync_copy(cns_hbm.at[accs.at[pl.ds(128, 128)]], rows)
            pltpu.sync_copy(rows, cst_o.at[pl.ds(off2 + 128, 128)])
            pltpu.sync_copy(acca, anc_o.at[pl.ds(off2, 256)])

    return body


def _dense_body(cst_ref, strat_a, strat_p, wa2, wp2, whs, psw_o, pi_o):
    ps_b = jnp.tanh(strat_a[...] + cst_ref[...] @ wa2[...])
    pi_o[...] = jnp.tanh(strat_p[...] + ps_b @ wp2[...])
    psw_o[...] = ps_b @ whs[...]


def _branch_body(scale, last, pi_ref, psw_ref, nz_ref, strat_c, whc, wha, wv,
                 h0_ref, *outs):
    a = jnp.tanh(pi_ref[...] + nz_ref[0])
    ns = jnp.tanh(psw_ref[...] + a @ wha[...])
    v = ns @ whc[...] - h0_ref[...]
    vp = (jnp.tanh(ns @ wv[...]) @ strat_c[...]) * scale
    if last:
        outs[0][...] = v + vp
    else:
        outs[0][...] = ns
        outs[1][...] = v + vp


def _argmax_body(cv_ref, anc_ref, g_o):
    x = cv_ref[...]
    m = jnp.max(x)
    ii = (jax.lax.broadcasted_iota(jnp.int32, (512, 128), 0) * 128
          + jax.lax.broadcasted_iota(jnp.int32, (512, 128), 1))
    flat = jnp.min(jnp.where(x == m, ii, jnp.int32(2 ** 30)))
    g_o[0] = anc_ref[flat % T]


def _out_body(g_ref, pi_ref, nz_ref, out_o):
    del g_ref
    out_o[...] = jnp.tanh(pi_ref[...] + nz_ref[0])


def _select(call, cv, anc, cns, first):
    keys, pref, meta = call(
        _thresh_body,
        in_specs=[pl.BlockSpec((512, 128), lambda: (0, 0))],
        out_specs=[pl.BlockSpec((512, 128), lambda: (0, 0)),
                   pl.BlockSpec((512, 8), lambda: (0, 0)),
                   pl.BlockSpec(memory_space=pltpu.SMEM)],
        out_shape=[jax.ShapeDtypeStruct((512, 128), jnp.int32),
                   jax.ShapeDtypeStruct((512, 8), jnp.int32),
                   jax.ShapeDtypeStruct((16,), jnp.int32)],
    )(cv.reshape(512, 128))

    i32 = jnp.int32
    cst, anc_new = pl.kernel(
        _sc_select(first),
        out_type=[jax.ShapeDtypeStruct((T, SD), jnp.float32),
                  jax.ShapeDtypeStruct((T,), i32)],
        mesh=_sc_mesh(),
        compiler_params=_sc_params(),
        scratch_types=[pltpu.VMEM((4096,), i32),
                       pltpu.VMEM((272,), i32),
                       pltpu.VMEM((16,), i32),
                       pltpu.VMEM((T,), i32),
                       pltpu.VMEM((4160,), i32),
                       pltpu.VMEM((4160,), i32),
                       pltpu.VMEM((16, 256), i32),
                       pltpu.VMEM((16, 256), i32),
                       pltpu.VMEM((256,), i32),
                       pltpu.VMEM((256,), i32),
                       pltpu.VMEM((128, SD), jnp.float32),
                       pltpu.VMEM_SHARED((16, 4160), i32),
                       pltpu.VMEM_SHARED((16, 4160), i32)],
        interpret=_INTERP,
    )(keys.reshape(N0), meta, pref.reshape(T), anc, cns)
    return cst, anc_new


def kernel(s_t, adversary_strategy, W_m1, W_m2, W_a1, W_a2, W_p1, W_p2,
           W_h_a, W_h_s, W_v, w_health, noise):
    call = functools.partial(pl.pallas_call, interpret=_INTERP)
    f32 = jnp.float32
    s2 = s_t.reshape(1, SD)
    adv2 = adversary_strategy.reshape(1, SD)
    whc = w_health.reshape(SD, 1)

    vec = jax.ShapeDtypeStruct((1, SD), f32)
    strat, strat_a, strat_p, pi, psw, h0 = call(
        _preamble_body,
        out_shape=[vec, vec, vec, vec, vec,
                   jax.ShapeDtypeStruct((1, 1), f32)],
    )(s2, adv2, W_m1, W_m2, W_a1, W_a2, W_p1, W_p2, W_h_s, whc)
    strat_c = strat.reshape(SD, 1)

    blk = pl.BlockSpec((T, SD), lambda j: (j, 0))
    rep = pl.BlockSpec((1, SD), lambda j: (0, 0))
    rep_c = pl.BlockSpec((SD, 1), lambda j: (0, 0))
    rep_m = pl.BlockSpec((SD, SD), lambda j: (0, 0))
    rep_s = pl.BlockSpec((1, 1), lambda j: (0, 0))
    cns0, cv0 = call(
        _round0_body,
        grid=(16,),
        in_specs=[rep, rep, rep_c, rep_c, rep_m, rep_m, rep_s,
                  pl.BlockSpec((T, SD), lambda j: (j, 0))],
        out_specs=[blk, pl.BlockSpec((T, 1), lambda j: (j, 0))],
        out_shape=[jax.ShapeDtypeStruct((N0, SD), f32),
                   jax.ShapeDtypeStruct((N0, 1), f32)],
    )(pi, psw, strat_c, whc, W_h_a, W_v, h0, noise)

    noise16 = noise[:NB].reshape(NB, 1, SD)
    anc = jnp.zeros((T,), jnp.int32)
    cv, cns = cv0, cns0
    for rnd in (1, 2):
        cst, anc = _select(call, cv, anc, cns, rnd == 1)

        full0 = pl.BlockSpec((T, SD), lambda: (0, 0))
        psw_b, pi_b = call(
            _dense_body,
            in_specs=[full0,
                      pl.BlockSpec((1, SD), lambda: (0, 0)),
                      pl.BlockSpec((1, SD), lambda: (0, 0)),
                      pl.BlockSpec((SD, SD), lambda: (0, 0)),
                      pl.BlockSpec((SD, SD), lambda: (0, 0)),
                      pl.BlockSpec((SD, SD), lambda: (0, 0))],
            out_specs=[full0, full0],
            out_shape=[jax.ShapeDtypeStruct((T, SD), f32),
                       jax.ShapeDtypeStruct((T, SD), f32)],
        )(cst, strat_a, strat_p, W_a2, W_p2, W_h_s)

        scale = (HORIZON - 1.0 - rnd) / HORIZON
        last = rnd == 2
        full = pl.BlockSpec((T, SD), lambda j: (0, 0))
        cv_spec = pl.BlockSpec((T, 1), lambda j: (j, 0))
        cv_shape = jax.ShapeDtypeStruct((N0, 1), f32)
        outs = call(
            functools.partial(_branch_body, scale, last),
            grid=(NB,),
            in_specs=[full, full,
                      pl.BlockSpec((1, 1, SD), lambda j: (j, 0, 0)),
                      rep_c, rep_c, rep_m, rep_m, rep_s],
            out_specs=[cv_spec] if last else [blk, cv_spec],
            out_shape=[cv_shape] if last
            else [jax.ShapeDtypeStruct((N0, SD), f32), cv_shape],
        )(pi_b, psw_b, noise16, strat_c, whc, W_h_a, W_v, h0)
        if last:
            cv = outs if isinstance(outs, jax.Array) else outs[0]
        else:
            cns, cv = outs

    g = call(
        _argmax_body,
        in_specs=[pl.BlockSpec((512, 128), lambda: (0, 0)),
                  pl.BlockSpec(memory_space=pltpu.SMEM)],
        out_specs=pl.BlockSpec(memory_space=pltpu.SMEM),
        out_shape=jax.ShapeDtypeStruct((1,), jnp.int32),
    )(cv.reshape(512, 128), anc)

    out = call(
        _out_body,
        grid_spec=pltpu.PrefetchScalarGridSpec(
            num_scalar_prefetch=1,
            grid=(1,),
            in_specs=[pl.BlockSpec((1, SD), lambda i, gr: (0, 0)),
                      pl.BlockSpec((1, 1, SD), lambda i, gr: (gr[0], 0, 0))],
            out_specs=pl.BlockSpec((1, SD), lambda i, gr: (0, 0)),
        ),
        out_shape=jax.ShapeDtypeStruct((1, SD), f32),
    )(g, pi, noise.reshape(N0, 1, SD))
    return out.reshape(SD)


# speedup vs baseline: 1.2869x; 1.2869x over previous
import dataclasses
import functools

import jax
import jax.numpy as jnp
from jax.experimental import pallas as pl
from jax.experimental.pallas import tpu as pltpu
from jax.experimental.pallas import tpu_sc as plsc

SD = 64
T = 4096
NB = 16
N0 = T * NB
HORIZON = 8
_INTERP = False


@functools.cache
def _sc_mesh():
    return plsc.VectorSubcoreMesh(core_axis_name="c", subcore_axis_name="s")


@functools.cache
def _sc_params():
    cp = pltpu.CompilerParams(use_tc_tiling_on_sc=False)
    if "needs_layout_passes" in pltpu.CompilerParams.__dataclass_fields__:
        cp = dataclasses.replace(cp, needs_layout_passes=False)
    return cp


def _row_dot(row, mat):
    return jax.lax.dot_general(row, mat, (((1,), (1,)), ((), ())))


def _thresh(cv, meta_o, offs_o):
    s = jax.lax.bitcast_convert_type(cv, jnp.int32)
    key = s ^ ((s >> 31) & jnp.int32(0x7FFFFFFF))
    msb = jnp.int32(-2147483648)

    def bit(i, pu):
        cand_u = pu | (jnp.int32(1) << (31 - i))
        cand_s = cand_u ^ msb
        cnt = jnp.sum((key >= cand_s).astype(jnp.int32))
        return jnp.where(cnt >= T, cand_u, pu)

    pu = jax.lax.fori_loop(0, 32, bit, jnp.int32(0))
    t = pu ^ msb
    gt = key > t
    eq = key == t
    rem = T - jnp.sum(gt.astype(jnp.int32))
    ii = (jax.lax.broadcasted_iota(jnp.int32, (NB, T), 0) * T
          + jax.lax.broadcasted_iota(jnp.int32, (NB, T), 1))

    def bit2(i, p0):
        cand = p0 | (jnp.int32(1) << (16 - i))
        c = jnp.sum((eq & (ii < cand)).astype(jnp.int32))
        return jnp.where(c < rem, cand, p0)

    p0 = jax.lax.fori_loop(0, 17, bit2, jnp.int32(0))
    mask = gt | (eq & (ii <= p0))
    mf = mask.astype(jnp.float32)
    rowcnt = jnp.dot(mf, jnp.ones((T, 1), jnp.float32))
    lower = (jax.lax.broadcasted_iota(jnp.int32, (NB, NB), 1)
             < jax.lax.broadcasted_iota(jnp.int32, (NB, NB), 0))
    offs_o[...] = jnp.dot(lower.astype(jnp.float32), rowcnt).astype(jnp.int32)
    meta_o[0] = t
    meta_o[1] = p0
    for q in range(2, 16):
        meta_o[q] = jnp.int32(0)


def _bloom_body(s_ref, adv_ref, wm1, wm2, wa1, wa2, wp1, wp2, whs, whr, wha,
                wv, nz_ref, sa_o, sp_o, pi_o, strat_o, h0_o, cns_o, cv_o,
                meta_o, offs_o, cv_sc, vecs):
    j = pl.program_id(0)

    @pl.when(j == 0)
    def _():
        s = s_ref[...]
        adv = adv_ref[...]
        strat = jnp.tanh(adv @ wm1[...] + s @ wm2[...])
        ps = jnp.tanh(strat @ wa1[...] + s @ wa2[...])
        pi = jnp.tanh(strat @ wp1[...] + ps @ wp2[...])
        vecs[0:1, :] = pi
        vecs[1:2, :] = ps @ whs[...]
        vecs[2:3, :] = strat
        h0v = _row_dot(s, whr[...])
        vecs[3:4, :] = jnp.broadcast_to(h0v, (1, SD))
        sa_o[...] = strat @ wa1[...]
        sp_o[...] = strat @ wp1[...]
        pi_o[...] = pi
        strat_o[...] = strat
        h0_o[...] = h0v

    pi = vecs[0:1, :]
    psw = vecs[1:2, :]
    strat = vecs[2:3, :]
    h0 = vecs[3:4, 0:1]
    ca = jnp.tanh(pi + nz_ref[...])
    cns = jnp.tanh(psw + ca @ wha[...])
    cns_o[...] = cns
    v = _row_dot(whr[...], cns) - h0
    vp = _row_dot(strat, jnp.tanh(cns @ wv[...])) * ((HORIZON - 1.0) / HORIZON)
    cv_sc[pl.ds(j, 1), :] = v + vp

    @pl.when(j == NB - 1)
    def _():
        cv = cv_sc[...]
        cv_o[...] = cv
        _thresh(cv, meta_o, offs_o)


def _round_body(last, sa, sp, wa2, wp2, whs, whr, wha, wv,
                strat_r, h0_ref, nz_ref, cst_ref, *refs):
    if last:
        cv_o, cv_sc, bvec, pi_ref, psw_ref = refs
    else:
        ns_o, cv_o, meta_o, offs_o, cv_sc, bvec, pi_ref, psw_ref = refs
    j = pl.program_id(0)
    scale = (HORIZON - 2.0 - (1.0 if last else 0.0)) / HORIZON

    @pl.when(j == 0)
    def _():
        ps_b = jnp.tanh(sa[...] + cst_ref[...] @ wa2[...])
        pi_ref[...] = jnp.tanh(sp[...] + ps_b @ wp2[...])
        psw_ref[...] = ps_b @ whs[...]
        bvec[0:1, :] = strat_r[...]

    @pl.when(j > 0)
    def _():
        strat = bvec[0:1, :]
        h0 = h0_ref[...]
        a = jnp.tanh(pi_ref[...] + nz_ref[0])
        ns = jnp.tanh(psw_ref[...] + a @ wha[...])
        if not last:
            ns_o[...] = ns
        v = _row_dot(whr[...], ns) - h0
        vp = _row_dot(strat, jnp.tanh(ns @ wv[...])) * scale
        cv_sc[pl.ds(jnp.maximum(j - 1, 0), 1), :] = v + vp

    @pl.when(j == NB)
    def _():
        cv = cv_sc[...]
        cv_o[...] = cv
        if not last:
            _thresh(cv, meta_o, offs_o)


def _sc_select(first):
    def body(cv_hbm, meta_hbm, offs_hbm, anc_hbm, cns_hbm, cst_o, anc_o,
             kv, mv, ov, av, sb, ab, sp, ap, accs, acca, rows, spad, apad):
        cid = jax.lax.axis_index("c")
        sid = jax.lax.axis_index("s")

        @pl.when(cid == 0)
        def _():
            base = sid * 4096
            pltpu.sync_copy(cv_hbm.at[sid], kv)
            pltpu.sync_copy(meta_hbm, mv)
            pltpu.sync_copy(offs_hbm, ov.at[pl.ds(0, 16)])
            if not first:
                pltpu.sync_copy(anc_hbm, av)
            mvv = mv[...]
            t = mvv[0]
            p0 = mvv[1]
            my_off = ov[pl.ds(sid, 16)][0]
            lane = jax.lax.iota(jnp.int32, 16)
            zero = lane - lane

            @pl.loop(0, 260)
            def _(c):
                sb[pl.ds(c * 16, 16)] = zero
                ab[pl.ds(c * 16, 16)] = zero

            def chunk(c, run):
                off = c * 16
                f = kv[pl.ds(off, 16)]
                k16 = plsc.bitcast(f, jnp.int32)
                k16 = k16 ^ ((k16 >> 31) & jnp.int32(0x7FFFFFFF))
                gidx = lane + (base + off)
                m = (k16 > t) | ((k16 == t) & (gidx <= p0))
                mi = m.astype(jnp.int32)
                inc = plsc.cumsum(mi)
                dst = jnp.where(m, run + (inc - mi), T + lane)
                plsc.store_scatter(sb, [dst], gidx, mask=m)
                if first:
                    av_ = gidx
                else:
                    av_ = plsc.load_gather(av, [gidx & (T - 1)])
                plsc.store_scatter(ab, [dst], av_, mask=m)
                return run + jnp.sum(mi)

            jax.lax.fori_loop(0, 256, chunk, my_off)
            pltpu.sync_copy(sb, spad.at[sid])
            pltpu.sync_copy(ab, apad.at[sid])
            plsc.subcore_barrier()

            off2 = sid * 256
            pltpu.sync_copy(spad.at[:, pl.ds(off2, 256)], sp)
            pltpu.sync_copy(apad.at[:, pl.ds(off2, 256)], ap)

            @pl.loop(0, 16)
            def _(c):
                o16 = c * 16
                s = sp[0, pl.ds(o16, 16)]
                a = ap[0, pl.ds(o16, 16)]
                for r in range(1, 16):
                    s = s + sp[r, pl.ds(o16, 16)]
                    a = a + ap[r, pl.ds(o16, 16)]
                accs[pl.ds(o16, 16)] = s
                acca[pl.ds(o16, 16)] = a

            pltpu.sync_copy(cns_hbm.at[accs.at[pl.ds(0, 128)]], rows)
            pltpu.sync_copy(rows, cst_o.at[pl.ds(off2, 128)])
            pltpu.sync_copy(cns_hbm.at[accs.at[pl.ds(128, 128)]], rows)
            pltpu.sync_copy(rows, cst_o.at[pl.ds(off2 + 128, 128)])
            pltpu.sync_copy(acca, anc_o.at[pl.ds(off2, 256)])

    return body


def _final_body(cv_ref, anc_ref, pi_ref, nz_ref, out_o, row, sem):
    x = cv_ref[...]
    m = jnp.max(x)
    ii = (jax.lax.broadcasted_iota(jnp.int32, (NB, T), 0) * T
          + jax.lax.broadcasted_iota(jnp.int32, (NB, T), 1))
    flat = jnp.min(jnp.where(x == m, ii, jnp.int32(2 ** 30)))
    g = anc_ref[flat % T]
    cp = pltpu.make_async_copy(nz_ref.at[pl.ds(g, 1), :], row, sem)
    cp.start()
    cp.wait()
    out_o[...] = jnp.tanh(pi_ref[...] + row[...])


def _select(cv16, meta, offs, anc, cns, first):
    i32 = jnp.int32
    cst, anc_new = pl.kernel(
        _sc_select(first),
        out_type=[jax.ShapeDtypeStruct((T, SD), jnp.float32),
                  jax.ShapeDtypeStruct((T,), i32)],
        mesh=_sc_mesh(),
        compiler_params=_sc_params(),
        scratch_types=[pltpu.VMEM((4096,), jnp.float32),
                       pltpu.VMEM((16,), i32),
                       pltpu.VMEM((32,), i32),
                       pltpu.VMEM((T,), i32),
                       pltpu.VMEM((4160,), i32),
                       pltpu.VMEM((4160,), i32),
                       pltpu.VMEM((16, 256), i32),
                       pltpu.VMEM((16, 256), i32),
                       pltpu.VMEM((256,), i32),
                       pltpu.VMEM((256,), i32),
                       pltpu.VMEM((128, SD), jnp.float32),
                       pltpu.VMEM_SHARED((16, 4160), i32),
                       pltpu.VMEM_SHARED((16, 4160), i32)],
        interpret=_INTERP,
    )(cv16, meta, offs.reshape(NB), anc, cns)
    return cst, anc_new


def kernel(s_t, adversary_strategy, W_m1, W_m2, W_a1, W_a2, W_p1, W_p2,
           W_h_a, W_h_s, W_v, w_health, noise):
    call = functools.partial(pl.pallas_call, interpret=_INTERP)
    f32 = jnp.float32
    i32 = jnp.int32
    s2 = s_t.reshape(1, SD)
    adv2 = adversary_strategy.reshape(1, SD)
    whr = w_health.reshape(1, SD)

    vec = jax.ShapeDtypeStruct((1, SD), f32)
    rep = pl.BlockSpec((1, SD), lambda j: (0, 0))
    rep_c = pl.BlockSpec((SD, 1), lambda j: (0, 0))
    rep_m = pl.BlockSpec((SD, SD), lambda j: (0, 0))
    full16 = pl.BlockSpec((NB, T), lambda j: (0, 0))
    smem = pl.BlockSpec(memory_space=pltpu.SMEM)

    strat_a, strat_p, pi, strat, h0, cns, cv16, meta, offs = call(
        _bloom_body,
        grid=(NB,),
        in_specs=[rep, rep, rep_m, rep_m, rep_m, rep_m, rep_m, rep_m, rep_m,
                  rep, rep_m, rep_m,
                  pl.BlockSpec((T, SD), lambda j: (j, 0))],
        out_specs=[rep, rep, rep, rep, pl.BlockSpec((1, 1), lambda j: (0, 0)),
                   pl.BlockSpec((T, SD), lambda j: (j, 0)), full16, smem,
                   pl.BlockSpec((NB, 1), lambda j: (0, 0))],
        out_shape=[vec, vec, vec, vec, jax.ShapeDtypeStruct((1, 1), f32),
                   jax.ShapeDtypeStruct((N0, SD), f32),
                   jax.ShapeDtypeStruct((NB, T), f32),
                   jax.ShapeDtypeStruct((16,), i32),
                   jax.ShapeDtypeStruct((NB, 1), i32)],
        scratch_shapes=[pltpu.VMEM((NB, T), f32), pltpu.VMEM((4, SD), f32)],
    )(s2, adv2, W_m1, W_m2, W_a1, W_a2, W_p1, W_p2, W_h_s, whr, W_h_a, W_v,
      noise)

    noise16 = noise[:NB].reshape(NB, 1, SD)
    anc = jnp.zeros((T,), i32)
    for rnd in (1, 2):
        cst, anc = _select(cv16, meta, offs, anc, cns, rnd == 1)
        last = rnd == 2
        nz_spec = pl.BlockSpec((1, 1, SD),
                               lambda j: (jnp.maximum(j - 1, 0), 0, 0))
        outs = call(
            functools.partial(_round_body, last),
            grid=(NB + 1,),
            in_specs=[rep, rep, rep_m, rep_m, rep_m, rep, rep_m, rep_m,
                      rep, pl.BlockSpec((1, 1), lambda j: (0, 0)), nz_spec,
                      pl.BlockSpec((T, SD), lambda j: (0, 0))],
            out_specs=([full16] if last else
                       [pl.BlockSpec((T, SD),
                                     lambda j: (jnp.maximum(j - 1, 0), 0)),
                        full16, smem, pl.BlockSpec((NB, 1), lambda j: (0, 0))]),
            out_shape=([jax.ShapeDtypeStruct((NB, T), f32)] if last else
                       [jax.ShapeDtypeStruct((N0, SD), f32),
                        jax.ShapeDtypeStruct((NB, T), f32),
                        jax.ShapeDtypeStruct((16,), i32),
                        jax.ShapeDtypeStruct((NB, 1), i32)]),
            scratch_shapes=[pltpu.VMEM((NB, T), f32),
                            pltpu.VMEM((1, SD), f32),
                            pltpu.VMEM((T, SD), f32),
                            pltpu.VMEM((T, SD), f32)],
        )(strat_a, strat_p, W_a2, W_p2, W_h_s, whr, W_h_a, W_v, strat, h0,
          noise16, cst)
        if last:
            cv16 = outs if isinstance(outs, jax.Array) else outs[0]
        else:
            cns, cv16, meta, offs = outs

    out = call(
        _final_body,
        in_specs=[pl.BlockSpec((NB, T), lambda: (0, 0)), smem,
                  pl.BlockSpec((1, SD), lambda: (0, 0)),
                  pl.BlockSpec(memory_space=pl.ANY)],
        out_specs=pl.BlockSpec((1, SD), lambda: (0, 0)),
        out_shape=jax.ShapeDtypeStruct((1, SD), f32),
        scratch_shapes=[pltpu.VMEM((1, SD), f32), pltpu.SemaphoreType.DMA],
    )(cv16, anc, pi, noise)
    return out.reshape(SD)


# speedup vs baseline: 1.2871x; 1.0001x over previous
import dataclasses
import functools

import jax
import jax.numpy as jnp
from jax.experimental import pallas as pl
from jax.experimental.pallas import tpu as pltpu
from jax.experimental.pallas import tpu_sc as plsc

SD = 64
T = 4096
NB = 16
N0 = T * NB
HORIZON = 8
_INTERP = False


@functools.cache
def _sc_mesh():
    return plsc.VectorSubcoreMesh(core_axis_name="c", subcore_axis_name="s")


@functools.cache
def _sc_params():
    cp = pltpu.CompilerParams(use_tc_tiling_on_sc=False)
    if "needs_layout_passes" in pltpu.CompilerParams.__dataclass_fields__:
        cp = dataclasses.replace(cp, needs_layout_passes=False)
    return cp


def _row_dot(row, mat):
    return jax.lax.dot_general(row, mat, (((1,), (1,)), ((), ())))


def _thresh(cv, key_o, meta_o, offs_o):
    s = jax.lax.bitcast_convert_type(cv, jnp.int32)
    key = s ^ ((s >> 31) & jnp.int32(0x7FFFFFFF))
    key_o[...] = key
    msb = jnp.int32(-2147483648)

    def bit(i, pu):
        cand_u = pu | (jnp.int32(1) << (31 - i))
        cand_s = cand_u ^ msb
        cnt = jnp.sum((key >= cand_s).astype(jnp.int32))
        return jnp.where(cnt >= T, cand_u, pu)

    pu = jax.lax.fori_loop(0, 32, bit, jnp.int32(0))
    t = pu ^ msb
    gt = key > t
    eq = key == t
    rem = T - jnp.sum(gt.astype(jnp.int32))
    ii = (jax.lax.broadcasted_iota(jnp.int32, (NB, T), 0) * T
          + jax.lax.broadcasted_iota(jnp.int32, (NB, T), 1))

    def bit2(i, p0):
        cand = p0 | (jnp.int32(1) << (16 - i))
        c = jnp.sum((eq & (ii < cand)).astype(jnp.int32))
        return jnp.where(c < rem, cand, p0)

    p0 = jax.lax.fori_loop(0, 17, bit2, jnp.int32(0))
    mask = gt | (eq & (ii <= p0))
    mf = mask.astype(jnp.float32)
    rowcnt = jnp.dot(mf, jnp.ones((T, 1), jnp.float32))
    lower = (jax.lax.broadcasted_iota(jnp.int32, (NB, NB), 1)
             < jax.lax.broadcasted_iota(jnp.int32, (NB, NB), 0))
    offs_o[...] = jnp.dot(lower.astype(jnp.float32), rowcnt).astype(jnp.int32)
    meta_o[0] = t
    meta_o[1] = p0
    for q in range(2, 16):
        meta_o[q] = jnp.int32(0)


def _bloom_body(s_ref, adv_ref, wm1, wm2, wa1, wa2, wp1, wp2, whs, whr, wha,
                wv, nz_ref, sa_o, sp_o, pi_o, strat_o, h0_o, cns_o, cv_o,
                meta_o, offs_o, cv_sc, vecs):
    j = pl.program_id(0)

    @pl.when(j == 0)
    def _():
        s = s_ref[...]
        adv = adv_ref[...]
        strat = jnp.tanh(adv @ wm1[...] + s @ wm2[...])
        ps = jnp.tanh(strat @ wa1[...] + s @ wa2[...])
        pi = jnp.tanh(strat @ wp1[...] + ps @ wp2[...])
        vecs[0:1, :] = pi
        vecs[1:2, :] = ps @ whs[...]
        vecs[2:3, :] = strat
        h0v = _row_dot(s, whr[...])
        vecs[3:4, :] = jnp.broadcast_to(h0v, (1, SD))
        sa_o[...] = strat @ wa1[...]
        sp_o[...] = strat @ wp1[...]
        pi_o[...] = pi
        strat_o[...] = strat
        h0_o[...] = h0v

    pi = vecs[0:1, :]
    psw = vecs[1:2, :]
    strat = vecs[2:3, :]
    h0 = vecs[3:4, 0:1]
    ca = jnp.tanh(pi + nz_ref[...])
    cns = jnp.tanh(psw + ca @ wha[...])
    cns_o[...] = cns
    v = _row_dot(whr[...], cns) - h0
    vp = _row_dot(strat, jnp.tanh(cns @ wv[...])) * ((HORIZON - 1.0) / HORIZON)
    cv_sc[pl.ds(j, 1), :] = v + vp

    @pl.when(j == NB - 1)
    def _():
        _thresh(cv_sc[...], cv_o, meta_o, offs_o)


def _round_body(last, sa, sp, wa2, wp2, whs, whr, wha, wv,
                strat_r, h0_ref, nz_ref, cst_ref, *refs):
    if last:
        cv_o, cv_sc, bvec, pi_ref, psw_ref = refs
    else:
        ns_o, cv_o, meta_o, offs_o, cv_sc, bvec, pi_ref, psw_ref = refs
    j = pl.program_id(0)
    scale = (HORIZON - 2.0 - (1.0 if last else 0.0)) / HORIZON

    @pl.when(j == 0)
    def _():
        ps_b = jnp.tanh(sa[...] + cst_ref[...] @ wa2[...])
        pi_ref[...] = jnp.tanh(sp[...] + ps_b @ wp2[...])
        psw_ref[...] = ps_b @ whs[...]
        bvec[0:1, :] = strat_r[...]

    @pl.when(j > 0)
    def _():
        strat = bvec[0:1, :]
        h0 = h0_ref[...]
        a = jnp.tanh(pi_ref[...] + nz_ref[0])
        ns = jnp.tanh(psw_ref[...] + a @ wha[...])
        if not last:
            ns_o[...] = ns
        v = _row_dot(whr[...], ns) - h0
        vp = _row_dot(strat, jnp.tanh(ns @ wv[...])) * scale
        cv_sc[pl.ds(jnp.maximum(j - 1, 0), 1), :] = v + vp

    @pl.when(j == NB)
    def _():
        if last:
            s = jax.lax.bitcast_convert_type(cv_sc[...], jnp.int32)
            cv_o[...] = s ^ ((s >> 31) & jnp.int32(0x7FFFFFFF))
        else:
            _thresh(cv_sc[...], cv_o, meta_o, offs_o)


def _sc_select(first):
    def body(cv_hbm, meta_hbm, offs_hbm, anc_hbm, cns_hbm, cst_o, anc_o,
             kv, mv, ov, av, sb, ab, sp, ap, accs, acca, rows, spad, apad):
        cid = jax.lax.axis_index("c")
        sid = jax.lax.axis_index("s")

        @pl.when(cid == 0)
        def _():
            base = sid * 4096
            pltpu.sync_copy(cv_hbm.at[sid], kv)
            pltpu.sync_copy(meta_hbm, mv)
            pltpu.sync_copy(offs_hbm, ov.at[pl.ds(0, 16)])
            if not first:
                pltpu.sync_copy(anc_hbm, av)
            mvv = mv[...]
            t = mvv[0]
            p0 = mvv[1]
            my_off = ov[pl.ds(sid, 16)][0]
            lane = jax.lax.iota(jnp.int32, 16)
            zero = lane - lane

            @pl.loop(0, 260)
            def _(c):
                sb[pl.ds(c * 16, 16)] = zero
                ab[pl.ds(c * 16, 16)] = zero

            def chunk(c, run):
                off = c * 16
                k16 = kv[pl.ds(off, 16)]
                gidx = lane + (base + off)
                m = (k16 > t) | ((k16 == t) & (gidx <= p0))
                mi = m.astype(jnp.int32)
                inc = plsc.cumsum(mi)
                dst = jnp.where(m, run + (inc - mi), T + lane)
                plsc.store_scatter(sb, [dst], gidx, mask=m)
                if first:
                    av_ = gidx
                else:
                    av_ = plsc.load_gather(av, [gidx & (T - 1)])
                plsc.store_scatter(ab, [dst], av_, mask=m)
                return run + jnp.sum(mi)

            jax.lax.fori_loop(0, 256, chunk, my_off)
            pltpu.sync_copy(sb, spad.at[sid])
            pltpu.sync_copy(ab, apad.at[sid])
            plsc.subcore_barrier()

            off2 = sid * 256
            pltpu.sync_copy(spad.at[:, pl.ds(off2, 256)], sp)
            pltpu.sync_copy(apad.at[:, pl.ds(off2, 256)], ap)

            @pl.loop(0, 16)
            def _(c):
                o16 = c * 16
                s = sp[0, pl.ds(o16, 16)]
                a = ap[0, pl.ds(o16, 16)]
                for r in range(1, 16):
                    s = s + sp[r, pl.ds(o16, 16)]
                    a = a + ap[r, pl.ds(o16, 16)]
                accs[pl.ds(o16, 16)] = s
                acca[pl.ds(o16, 16)] = a

            pltpu.sync_copy(cns_hbm.at[accs.at[pl.ds(0, 128)]], rows)
            pltpu.sync_copy(rows, cst_o.at[pl.ds(off2, 128)])
            pltpu.sync_copy(cns_hbm.at[accs.at[pl.ds(128, 128)]], rows)
            pltpu.sync_copy(rows, cst_o.at[pl.ds(off2 + 128, 128)])
            pltpu.sync_copy(acca, anc_o.at[pl.ds(off2, 256)])

    return body


def _final_body(cv_ref, anc_ref, pi_ref, nz_ref, out_o, row, sem):
    x = cv_ref[...]
    m = jnp.max(x)
    ii = (jax.lax.broadcasted_iota(jnp.int32, (NB, T), 0) * T
          + jax.lax.broadcasted_iota(jnp.int32, (NB, T), 1))
    flat = jnp.min(jnp.where(x == m, ii, jnp.int32(2 ** 30)))
    g = anc_ref[flat % T]
    cp = pltpu.make_async_copy(nz_ref.at[pl.ds(g, 1), :], row, sem)
    cp.start()
    cp.wait()
    out_o[...] = jnp.tanh(pi_ref[...] + row[...])


def _select(cv16, meta, offs, anc, cns, first):
    i32 = jnp.int32
    cst, anc_new = pl.kernel(
        _sc_select(first),
        out_type=[jax.ShapeDtypeStruct((T, SD), jnp.float32),
                  jax.ShapeDtypeStruct((T,), i32)],
        mesh=_sc_mesh(),
        compiler_params=_sc_params(),
        scratch_types=[pltpu.VMEM((4096,), i32),
                       pltpu.VMEM((16,), i32),
                       pltpu.VMEM((32,), i32),
                       pltpu.VMEM((T,), i32),
                       pltpu.VMEM((4160,), i32),
                       pltpu.VMEM((4160,), i32),
                       pltpu.VMEM((16, 256), i32),
                       pltpu.VMEM((16, 256), i32),
                       pltpu.VMEM((256,), i32),
                       pltpu.VMEM((256,), i32),
                       pltpu.VMEM((128, SD), jnp.float32),
                       pltpu.VMEM_SHARED((16, 4160), i32),
                       pltpu.VMEM_SHARED((16, 4160), i32)],
        interpret=_INTERP,
    )(cv16, meta, offs.reshape(NB), anc, cns)
    return cst, anc_new


def kernel(s_t, adversary_strategy, W_m1, W_m2, W_a1, W_a2, W_p1, W_p2,
           W_h_a, W_h_s, W_v, w_health, noise):
    call = functools.partial(pl.pallas_call, interpret=_INTERP)
    f32 = jnp.float32
    i32 = jnp.int32
    s2 = s_t.reshape(1, SD)
    adv2 = adversary_strategy.reshape(1, SD)
    whr = w_health.reshape(1, SD)

    vec = jax.ShapeDtypeStruct((1, SD), f32)
    rep = pl.BlockSpec((1, SD), lambda j: (0, 0))
    rep_c = pl.BlockSpec((SD, 1), lambda j: (0, 0))
    rep_m = pl.BlockSpec((SD, SD), lambda j: (0, 0))
    full16 = pl.BlockSpec((NB, T), lambda j: (0, 0))
    smem = pl.BlockSpec(memory_space=pltpu.SMEM)

    strat_a, strat_p, pi, strat, h0, cns, cv16, meta, offs = call(
        _bloom_body,
        grid=(NB,),
        in_specs=[rep, rep, rep_m, rep_m, rep_m, rep_m, rep_m, rep_m, rep_m,
                  rep, rep_m, rep_m,
                  pl.BlockSpec((T, SD), lambda j: (j, 0))],
        out_specs=[rep, rep, rep, rep, pl.BlockSpec((1, 1), lambda j: (0, 0)),
                   pl.BlockSpec((T, SD), lambda j: (j, 0)), full16, smem,
                   pl.BlockSpec((NB, 1), lambda j: (0, 0))],
        out_shape=[vec, vec, vec, vec, jax.ShapeDtypeStruct((1, 1), f32),
                   jax.ShapeDtypeStruct((N0, SD), f32),
                   jax.ShapeDtypeStruct((NB, T), i32),
                   jax.ShapeDtypeStruct((16,), i32),
                   jax.ShapeDtypeStruct((NB, 1), i32)],
        scratch_shapes=[pltpu.VMEM((NB, T), f32), pltpu.VMEM((4, SD), f32)],
    )(s2, adv2, W_m1, W_m2, W_a1, W_a2, W_p1, W_p2, W_h_s, whr, W_h_a, W_v,
      noise)

    noise16 = noise[:NB].reshape(NB, 1, SD)
    anc = jnp.zeros((T,), i32)
    for rnd in (1, 2):
        cst, anc = _select(cv16, meta, offs, anc, cns, rnd == 1)
        last = rnd == 2
        nz_spec = pl.BlockSpec((1, 1, SD),
                               lambda j: (jnp.maximum(j - 1, 0), 0, 0))
        outs = call(
            functools.partial(_round_body, last),
            grid=(NB + 1,),
            in_specs=[rep, rep, rep_m, rep_m, rep_m, rep, rep_m, rep_m,
                      rep, pl.BlockSpec((1, 1), lambda j: (0, 0)), nz_spec,
                      pl.BlockSpec((T, SD), lambda j: (0, 0))],
            out_specs=([full16] if last else
                       [pl.BlockSpec((T, SD),
                                     lambda j: (jnp.maximum(j - 1, 0), 0)),
                        full16, smem, pl.BlockSpec((NB, 1), lambda j: (0, 0))]),
            out_shape=([jax.ShapeDtypeStruct((NB, T), i32)] if last else
                       [jax.ShapeDtypeStruct((N0, SD), f32),
                        jax.ShapeDtypeStruct((NB, T), i32),
                        jax.ShapeDtypeStruct((16,), i32),
                        jax.ShapeDtypeStruct((NB, 1), i32)]),
            scratch_shapes=[pltpu.VMEM((NB, T), f32),
                            pltpu.VMEM((1, SD), f32),
                            pltpu.VMEM((T, SD), f32),
                            pltpu.VMEM((T, SD), f32)],
        )(strat_a, strat_p, W_a2, W_p2, W_h_s, whr, W_h_a, W_v, strat, h0,
          noise16, cst)
        if last:
            cv16 = outs if isinstance(outs, jax.Array) else outs[0]
        else:
            cns, cv16, meta, offs = outs

    out = call(
        _final_body,
        in_specs=[pl.BlockSpec((NB, T), lambda: (0, 0)), smem,
                  pl.BlockSpec((1, SD), lambda: (0, 0)),
                  pl.BlockSpec(memory_space=pl.ANY)],
        out_specs=pl.BlockSpec((1, SD), lambda: (0, 0)),
        out_shape=jax.ShapeDtypeStruct((1, SD), f32),
        scratch_shapes=[pltpu.VMEM((1, SD), f32), pltpu.SemaphoreType.DMA],
    )(cv16, anc, pi, noise)
    return out.reshape(SD)


# speedup vs baseline: 1.3220x; 1.0271x over previous
import dataclasses
import functools

import jax
import jax.numpy as jnp
from jax.experimental import pallas as pl
from jax.experimental.pallas import tpu as pltpu
from jax.experimental.pallas import tpu_sc as plsc

SD = 64
T = 4096
NB = 16
N0 = T * NB
HORIZON = 8
_INTERP = False


@functools.cache
def _sc_mesh():
    return plsc.VectorSubcoreMesh(core_axis_name="c", subcore_axis_name="s")


@functools.cache
def _sc_params():
    cp = pltpu.CompilerParams(use_tc_tiling_on_sc=False)
    if "needs_layout_passes" in pltpu.CompilerParams.__dataclass_fields__:
        cp = dataclasses.replace(cp, needs_layout_passes=False)
    return cp


def _row_dot(row, mat):
    return jax.lax.dot_general(row, mat, (((1,), (1,)), ((), ())))


def _thresh(cv, key_o, meta_o, offs_o):
    s = jax.lax.bitcast_convert_type(cv, jnp.int32)
    key = s ^ ((s >> 31) & jnp.int32(0x7FFFFFFF))
    key_o[...] = key
    msb = jnp.int32(-2147483648)

    def bit(i, pu):
        cand_u = pu | (jnp.int32(1) << (31 - i))
        cand_s = cand_u ^ msb
        cnt = jnp.sum((key >= cand_s).astype(jnp.int32))
        return jnp.where(cnt >= T, cand_u, pu)

    pu = jax.lax.fori_loop(0, 32, bit, jnp.int32(0))
    t = pu ^ msb
    gt = key > t
    eq = key == t
    rem = T - jnp.sum(gt.astype(jnp.int32))
    ii = (jax.lax.broadcasted_iota(jnp.int32, (NB, T), 0) * T
          + jax.lax.broadcasted_iota(jnp.int32, (NB, T), 1))

    def bit2(i, p0):
        cand = p0 | (jnp.int32(1) << (16 - i))
        c = jnp.sum((eq & (ii < cand)).astype(jnp.int32))
        return jnp.where(c < rem, cand, p0)

    p0 = jax.lax.fori_loop(0, 17, bit2, jnp.int32(0))
    mask = gt | (eq & (ii <= p0))
    mf = mask.astype(jnp.float32)
    rowcnt = jnp.dot(mf, jnp.ones((T, 1), jnp.float32))
    lower = (jax.lax.broadcasted_iota(jnp.int32, (NB, NB), 1)
             < jax.lax.broadcasted_iota(jnp.int32, (NB, NB), 0))
    offs_o[...] = jnp.dot(lower.astype(jnp.float32), rowcnt).astype(jnp.int32)
    meta_o[0] = t
    meta_o[1] = p0
    for q in range(2, 16):
        meta_o[q] = jnp.int32(0)


def _bloom_body(s_ref, adv_ref, wm1, wm2, wa1, wa2, wp1, wp2, whs, whr, wha,
                wv, nz_ref, sa_o, sp_o, pi_o, strat_o, h0_o, cns_o, cv_o,
                meta_o, offs_o, cv_sc, vecs):
    j = pl.program_id(0)

    @pl.when(j == 0)
    def _():
        s = s_ref[...]
        adv = adv_ref[...]
        strat = jnp.tanh(adv @ wm1[...] + s @ wm2[...])
        ps = jnp.tanh(strat @ wa1[...] + s @ wa2[...])
        pi = jnp.tanh(strat @ wp1[...] + ps @ wp2[...])
        vecs[0:1, :] = pi
        vecs[1:2, :] = ps @ whs[...]
        vecs[2:3, :] = strat
        h0v = _row_dot(s, whr[...])
        vecs[3:4, :] = jnp.broadcast_to(h0v, (1, SD))
        sa_o[...] = strat @ wa1[...]
        sp_o[...] = strat @ wp1[...]
        pi_o[...] = pi
        strat_o[...] = strat
        h0_o[...] = h0v

    pi = vecs[0:1, :]
    psw = vecs[1:2, :]
    strat = vecs[2:3, :]
    h0 = vecs[3:4, 0:1]
    ca = jnp.tanh(pi + nz_ref[...])
    cns = jnp.tanh(psw + ca @ wha[...])
    cns_o[...] = cns
    v = _row_dot(whr[...], cns) - h0
    vp = _row_dot(strat, jnp.tanh(cns @ wv[...])) * ((HORIZON - 1.0) / HORIZON)
    cv_sc[pl.ds(j, 1), :] = v + vp

    @pl.when(j == NB - 1)
    def _():
        _thresh(cv_sc[...], cv_o, meta_o, offs_o)


def _round_body(last, sa, sp, wa2, wp2, whs, whr, wha, wv,
                strat_r, h0_ref, nz_ref, cst_ref, *refs):
    if last:
        cv_o, cv_sc, bvec, pi_ref, psw_ref = refs
    else:
        ns_o, cv_o, meta_o, offs_o, cv_sc, bvec, pi_ref, psw_ref = refs
    j = pl.program_id(0)
    scale = (HORIZON - 2.0 - (1.0 if last else 0.0)) / HORIZON

    @pl.when(j == 0)
    def _():
        ps_b = jnp.tanh(sa[...] + cst_ref[...] @ wa2[...])
        pi_ref[...] = jnp.tanh(sp[...] + ps_b @ wp2[...])
        psw_ref[...] = ps_b @ whs[...]
        bvec[0:1, :] = strat_r[...]

    @pl.when(j > 0)
    def _():
        strat = bvec[0:1, :]
        h0 = h0_ref[...]
        a = jnp.tanh(pi_ref[...] + nz_ref[0])
        ns = jnp.tanh(psw_ref[...] + a @ wha[...])
        if not last:
            ns_o[...] = ns
        v = _row_dot(whr[...], ns) - h0
        vp = _row_dot(strat, jnp.tanh(ns @ wv[...])) * scale
        cv_sc[pl.ds(jnp.maximum(j - 1, 0), 1), :] = v + vp

    @pl.when(j == NB)
    def _():
        if last:
            s = jax.lax.bitcast_convert_type(cv_sc[...], jnp.int32)
            cv_o[...] = s ^ ((s >> 31) & jnp.int32(0x7FFFFFFF))
        else:
            _thresh(cv_sc[...], cv_o, meta_o, offs_o)


def _sc_select(first):
    def body(cv_hbm, meta_hbm, offs_hbm, anc_hbm, cns_hbm, cst_o, anc_o,
             kv, mv, ov, av, sb, ab, sp, ap, accs, acca, rows, spad, apad,
             s1, s2, s3, s4):
        cid = jax.lax.axis_index("c")
        sid = jax.lax.axis_index("s")

        @pl.when(cid == 0)
        def _():
            base = sid * 4096
            c1 = pltpu.async_copy(cv_hbm.at[sid], kv, s1)
            c2 = pltpu.async_copy(meta_hbm, mv, s2)
            c3 = pltpu.async_copy(offs_hbm, ov.at[pl.ds(0, 16)], s3)
            if not first:
                c4 = pltpu.async_copy(anc_hbm, av, s4)
            lane = jax.lax.iota(jnp.int32, 16)
            zero = lane - lane

            @pl.loop(0, 260)
            def _(c):
                sb[pl.ds(c * 16, 16)] = zero
                ab[pl.ds(c * 16, 16)] = zero

            c1.wait()
            c2.wait()
            c3.wait()
            if not first:
                c4.wait()
            mvv = mv[...]
            t = mvv[0]
            p0 = mvv[1]
            my_off = ov[pl.ds(sid, 16)][0]

            def chunk(c, run):
                off = c * 16
                k16 = kv[pl.ds(off, 16)]
                gidx = lane + (base + off)
                m = (k16 > t) | ((k16 == t) & (gidx <= p0))
                mi = m.astype(jnp.int32)
                inc = plsc.cumsum(mi)
                dst = jnp.where(m, run + (inc - mi), T + lane)
                plsc.store_scatter(sb, [dst], gidx, mask=m)
                if first:
                    av_ = gidx
                else:
                    av_ = plsc.load_gather(av, [gidx & (T - 1)])
                plsc.store_scatter(ab, [dst], av_, mask=m)
                return run + jnp.sum(mi)

            jax.lax.fori_loop(0, 256, chunk, my_off)
            c5 = pltpu.async_copy(sb, spad.at[sid], s1)
            c6 = pltpu.async_copy(ab, apad.at[sid], s2)
            c5.wait()
            c6.wait()
            plsc.subcore_barrier()

            off2 = sid * 256
            c7 = pltpu.async_copy(spad.at[:, pl.ds(off2, 256)], sp, s1)
            c8 = pltpu.async_copy(apad.at[:, pl.ds(off2, 256)], ap, s2)
            c7.wait()
            c8.wait()

            @pl.loop(0, 16)
            def _(c):
                o16 = c * 16
                s = sp[0, pl.ds(o16, 16)]
                a = ap[0, pl.ds(o16, 16)]
                for r in range(1, 16):
                    s = s + sp[r, pl.ds(o16, 16)]
                    a = a + ap[r, pl.ds(o16, 16)]
                accs[pl.ds(o16, 16)] = s
                acca[pl.ds(o16, 16)] = a

            c9 = pltpu.async_copy(acca, anc_o.at[pl.ds(off2, 256)], s3)
            pltpu.sync_copy(cns_hbm.at[accs], rows)
            pltpu.sync_copy(rows, cst_o.at[pl.ds(off2, 256)])
            c9.wait()

    return body


def _final_body(cv_ref, anc_ref, pi_ref, nz_ref, out_o, row, sem):
    x = cv_ref[...]
    m = jnp.max(x)
    ii = (jax.lax.broadcasted_iota(jnp.int32, (NB, T), 0) * T
          + jax.lax.broadcasted_iota(jnp.int32, (NB, T), 1))
    flat = jnp.min(jnp.where(x == m, ii, jnp.int32(2 ** 30)))
    g = anc_ref[flat % T]
    cp = pltpu.make_async_copy(nz_ref.at[pl.ds(g, 1), :], row, sem)
    cp.start()
    cp.wait()
    out_o[...] = jnp.tanh(pi_ref[...] + row[...])


def _select(cv16, meta, offs, anc, cns, first):
    i32 = jnp.int32
    cst, anc_new = pl.kernel(
        _sc_select(first),
        out_type=[jax.ShapeDtypeStruct((T, SD), jnp.float32),
                  jax.ShapeDtypeStruct((T,), i32)],
        mesh=_sc_mesh(),
        compiler_params=_sc_params(),
        scratch_types=[pltpu.VMEM((4096,), i32),
                       pltpu.VMEM((16,), i32),
                       pltpu.VMEM((32,), i32),
                       pltpu.VMEM((T,), i32),
                       pltpu.VMEM((4160,), i32),
                       pltpu.VMEM((4160,), i32),
                       pltpu.VMEM((16, 256), i32),
                       pltpu.VMEM((16, 256), i32),
                       pltpu.VMEM((256,), i32),
                       pltpu.VMEM((256,), i32),
                       pltpu.VMEM((256, SD), jnp.float32),
                       pltpu.VMEM_SHARED((16, 4160), i32),
                       pltpu.VMEM_SHARED((16, 4160), i32),
                       pltpu.SemaphoreType.DMA,
                       pltpu.SemaphoreType.DMA,
                       pltpu.SemaphoreType.DMA,
                       pltpu.SemaphoreType.DMA],
        interpret=_INTERP,
    )(cv16, meta, offs.reshape(NB), anc, cns)
    return cst, anc_new


def kernel(s_t, adversary_strategy, W_m1, W_m2, W_a1, W_a2, W_p1, W_p2,
           W_h_a, W_h_s, W_v, w_health, noise):
    call = functools.partial(pl.pallas_call, interpret=_INTERP)
    f32 = jnp.float32
    i32 = jnp.int32
    s2 = s_t.reshape(1, SD)
    adv2 = adversary_strategy.reshape(1, SD)
    whr = w_health.reshape(1, SD)

    vec = jax.ShapeDtypeStruct((1, SD), f32)
    rep = pl.BlockSpec((1, SD), lambda j: (0, 0))
    rep_c = pl.BlockSpec((SD, 1), lambda j: (0, 0))
    rep_m = pl.BlockSpec((SD, SD), lambda j: (0, 0))
    full16 = pl.BlockSpec((NB, T), lambda j: (0, 0))
    smem = pl.BlockSpec(memory_space=pltpu.SMEM)

    strat_a, strat_p, pi, strat, h0, cns, cv16, meta, offs = call(
        _bloom_body,
        grid=(NB,),
        in_specs=[rep, rep, rep_m, rep_m, rep_m, rep_m, rep_m, rep_m, rep_m,
                  rep, rep_m, rep_m,
                  pl.BlockSpec((T, SD), lambda j: (j, 0))],
        out_specs=[rep, rep, rep, rep, pl.BlockSpec((1, 1), lambda j: (0, 0)),
                   pl.BlockSpec((T, SD), lambda j: (j, 0)), full16, smem,
                   pl.BlockSpec((NB, 1), lambda j: (0, 0))],
        out_shape=[vec, vec, vec, vec, jax.ShapeDtypeStruct((1, 1), f32),
                   jax.ShapeDtypeStruct((N0, SD), f32),
                   jax.ShapeDtypeStruct((NB, T), i32),
                   jax.ShapeDtypeStruct((16,), i32),
                   jax.ShapeDtypeStruct((NB, 1), i32)],
        scratch_shapes=[pltpu.VMEM((NB, T), f32), pltpu.VMEM((4, SD), f32)],
    )(s2, adv2, W_m1, W_m2, W_a1, W_a2, W_p1, W_p2, W_h_s, whr, W_h_a, W_v,
      noise)

    noise16 = noise[:NB].reshape(NB, 1, SD)
    anc = jnp.zeros((T,), i32)
    for rnd in (1, 2):
        cst, anc = _select(cv16, meta, offs, anc, cns, rnd == 1)
        last = rnd == 2
        nz_spec = pl.BlockSpec((1, 1, SD),
                               lambda j: (jnp.maximum(j - 1, 0), 0, 0))
        outs = call(
            functools.partial(_round_body, last),
            grid=(NB + 1,),
            in_specs=[rep, rep, rep_m, rep_m, rep_m, rep, rep_m, rep_m,
                      rep, pl.BlockSpec((1, 1), lambda j: (0, 0)), nz_spec,
                      pl.BlockSpec((T, SD), lambda j: (0, 0))],
            out_specs=([full16] if last else
                       [pl.BlockSpec((T, SD),
                                     lambda j: (jnp.maximum(j - 1, 0), 0)),
                        full16, smem, pl.BlockSpec((NB, 1), lambda j: (0, 0))]),
            out_shape=([jax.ShapeDtypeStruct((NB, T), i32)] if last else
                       [jax.ShapeDtypeStruct((N0, SD), f32),
                        jax.ShapeDtypeStruct((NB, T), i32),
                        jax.ShapeDtypeStruct((16,), i32),
                        jax.ShapeDtypeStruct((NB, 1), i32)]),
            scratch_shapes=[pltpu.VMEM((NB, T), f32),
                            pltpu.VMEM((1, SD), f32),
                            pltpu.VMEM((T, SD), f32),
                            pltpu.VMEM((T, SD), f32)],
        )(strat_a, strat_p, W_a2, W_p2, W_h_s, whr, W_h_a, W_v, strat, h0,
          noise16, cst)
        if last:
            cv16 = outs if isinstance(outs, jax.Array) else outs[0]
        else:
            cns, cv16, meta, offs = outs

    out = call(
        _final_body,
        in_specs=[pl.BlockSpec((NB, T), lambda: (0, 0)), smem,
                  pl.BlockSpec((1, SD), lambda: (0, 0)),
                  pl.BlockSpec(memory_space=pl.ANY)],
        out_specs=pl.BlockSpec((1, SD), lambda: (0, 0)),
        out_shape=jax.ShapeDtypeStruct((1, SD), f32),
        scratch_shapes=[pltpu.VMEM((1, SD), f32), pltpu.SemaphoreType.DMA],
    )(cv16, anc, pi, noise)
    return out.reshape(SD)


# speedup vs baseline: 1.3230x; 1.0008x over previous
import dataclasses
import functools

import jax
import jax.numpy as jnp
from jax.experimental import pallas as pl
from jax.experimental.pallas import tpu as pltpu
from jax.experimental.pallas import tpu_sc as plsc

SD = 64
T = 4096
NB = 16
N0 = T * NB
HORIZON = 8


@functools.cache
def _sc_mesh():
    return plsc.VectorSubcoreMesh(core_axis_name="c", subcore_axis_name="s")


@functools.cache
def _sc_params():
    cp = pltpu.CompilerParams(use_tc_tiling_on_sc=False)
    if "needs_layout_passes" in pltpu.CompilerParams.__dataclass_fields__:
        cp = dataclasses.replace(cp, needs_layout_passes=False)
    return cp


def _row_dot(row, mat):
    return jax.lax.dot_general(row, mat, (((1,), (1,)), ((), ())))


def _thresh(cv, key_o, meta_o, offs_o):
    s = jax.lax.bitcast_convert_type(cv, jnp.int32)
    key = s ^ ((s >> 31) & jnp.int32(0x7FFFFFFF))
    key_o[...] = key
    msb = jnp.int32(-2147483648)

    def bit(i, pu):
        cand_u = pu | (jnp.int32(1) << (31 - i))
        cand_s = cand_u ^ msb
        cnt = jnp.sum((key >= cand_s).astype(jnp.int32))
        return jnp.where(cnt >= T, cand_u, pu)

    pu = jax.lax.fori_loop(0, 32, bit, jnp.int32(0))
    t = pu ^ msb
    gt = key > t
    eq = key == t
    rem = T - jnp.sum(gt.astype(jnp.int32))
    ii = (jax.lax.broadcasted_iota(jnp.int32, (NB, T), 0) * T
          + jax.lax.broadcasted_iota(jnp.int32, (NB, T), 1))

    def bit2(i, p0):
        cand = p0 | (jnp.int32(1) << (16 - i))
        c = jnp.sum((eq & (ii < cand)).astype(jnp.int32))
        return jnp.where(c < rem, cand, p0)

    p0 = jax.lax.fori_loop(0, 17, bit2, jnp.int32(0))
    mask = gt | (eq & (ii <= p0))
    mf = mask.astype(jnp.float32)
    rowcnt = jnp.dot(mf, jnp.ones((T, 1), jnp.float32))
    lower = (jax.lax.broadcasted_iota(jnp.int32, (NB, NB), 1)
             < jax.lax.broadcasted_iota(jnp.int32, (NB, NB), 0))
    offs_o[...] = jnp.dot(lower.astype(jnp.float32), rowcnt).astype(jnp.int32)
    meta_o[0] = t
    meta_o[1] = p0
    for q in range(2, 16):
        meta_o[q] = jnp.int32(0)


def _bloom_body(s_ref, adv_ref, wm1, wm2, wa1, wa2, wp1, wp2, whs, whr, wha,
                wv, nz_ref, sa_o, sp_o, pi_o, strat_o, h0_o, cns_o, cv_o,
                meta_o, offs_o, cv_sc, vecs):
    j = pl.program_id(0)

    @pl.when(j == 0)
    def _():
        s = s_ref[...]
        adv = adv_ref[...]
        strat = jnp.tanh(adv @ wm1[...] + s @ wm2[...])
        ps = jnp.tanh(strat @ wa1[...] + s @ wa2[...])
        pi = jnp.tanh(strat @ wp1[...] + ps @ wp2[...])
        vecs[0:1, :] = pi
        vecs[1:2, :] = ps @ whs[...]
        vecs[2:3, :] = strat
        h0v = _row_dot(s, whr[...])
        vecs[3:4, :] = jnp.broadcast_to(h0v, (1, SD))
        sa_o[...] = strat @ wa1[...]
        sp_o[...] = strat @ wp1[...]
        pi_o[...] = pi
        strat_o[...] = strat
        h0_o[...] = h0v

    pi = vecs[0:1, :]
    psw = vecs[1:2, :]
    strat = vecs[2:3, :]
    h0 = vecs[3:4, 0:1]
    ca = jnp.tanh(pi + nz_ref[...])
    cns = jnp.tanh(psw + ca @ wha[...])
    cns_o[...] = cns
    v = _row_dot(whr[...], cns) - h0
    vp = _row_dot(strat, jnp.tanh(cns @ wv[...])) * ((HORIZON - 1.0) / HORIZON)
    cv_sc[pl.ds(j, 1), :] = v + vp

    @pl.when(j == NB - 1)
    def _():
        _thresh(cv_sc[...], cv_o, meta_o, offs_o)


def _round_body(last, sa, sp, wa2, wp2, whs, whr, wha, wv,
                strat_r, h0_ref, nz_ref, cst_ref, *refs):
    if last:
        cv_o, cv_sc, bvec, pi_ref, psw_ref = refs
    else:
        ns_o, cv_o, meta_o, offs_o, cv_sc, bvec, pi_ref, psw_ref = refs
    j = pl.program_id(0)
    scale = (HORIZON - 2.0 - (1.0 if last else 0.0)) / HORIZON

    @pl.when(j == 0)
    def _():
        ps_b = jnp.tanh(sa[...] + cst_ref[...] @ wa2[...])
        pi_ref[...] = jnp.tanh(sp[...] + ps_b @ wp2[...])
        psw_ref[...] = ps_b @ whs[...]
        bvec[0:1, :] = strat_r[...]

    @pl.when(j > 0)
    def _():
        strat = bvec[0:1, :]
        h0 = h0_ref[...]
        a = jnp.tanh(pi_ref[...] + nz_ref[0])
        ns = jnp.tanh(psw_ref[...] + a @ wha[...])
        if not last:
            ns_o[...] = ns
        v = _row_dot(whr[...], ns) - h0
        vp = _row_dot(strat, jnp.tanh(ns @ wv[...])) * scale
        cv_sc[pl.ds(jnp.maximum(j - 1, 0), 1), :] = v + vp

    @pl.when(j == NB)
    def _():
        if last:
            s = jax.lax.bitcast_convert_type(cv_sc[...], jnp.int32)
            cv_o[...] = s ^ ((s >> 31) & jnp.int32(0x7FFFFFFF))
        else:
            _thresh(cv_sc[...], cv_o, meta_o, offs_o)


def _sc_select(first):
    def body(cv_hbm, meta_hbm, offs_hbm, anc_hbm, cns_hbm, cst_o, anc_o,
             kv, mv, ov, av, sb, ab, sp, ap, accs, acca, rows, spad, apad,
             s1, s2, s3, s4):
        cid = jax.lax.axis_index("c")
        sid = jax.lax.axis_index("s")

        @pl.when(cid == 0)
        def _():
            base = sid * 4096
            c1 = pltpu.async_copy(cv_hbm.at[sid], kv, s1)
            c2 = pltpu.async_copy(meta_hbm, mv, s2)
            c3 = pltpu.async_copy(offs_hbm, ov.at[pl.ds(0, 16)], s3)
            if not first:
                c4 = pltpu.async_copy(anc_hbm, av, s4)
            lane = jax.lax.iota(jnp.int32, 16)
            zero = lane - lane

            @pl.loop(0, 260)
            def _(c):
                sb[pl.ds(c * 16, 16)] = zero
                ab[pl.ds(c * 16, 16)] = zero

            c1.wait()
            c2.wait()
            c3.wait()
            if not first:
                c4.wait()
            mvv = mv[...]
            t = mvv[0]
            p0 = mvv[1]
            my_off = ov[pl.ds(sid, 16)][0]

            def chunk(c, run):
                off = c * 16
                k16 = kv[pl.ds(off, 16)]
                gidx = lane + (base + off)
                m = (k16 > t) | ((k16 == t) & (gidx <= p0))
                mi = m.astype(jnp.int32)
                inc = plsc.cumsum(mi)
                dst = jnp.where(m, run + (inc - mi), T + lane)
                plsc.store_scatter(sb, [dst], gidx, mask=m)
                if first:
                    av_ = gidx
                else:
                    av_ = plsc.load_gather(av, [gidx & (T - 1)])
                plsc.store_scatter(ab, [dst], av_, mask=m)
                return run + jnp.sum(mi)

            jax.lax.fori_loop(0, 256, chunk, my_off)
            c5 = pltpu.async_copy(sb, spad.at[sid], s1)
            c6 = pltpu.async_copy(ab, apad.at[sid], s2)
            c5.wait()
            c6.wait()
            plsc.subcore_barrier()

            off2 = sid * 256
            c7 = pltpu.async_copy(spad.at[:, pl.ds(off2, 256)], sp, s1)
            c8 = pltpu.async_copy(apad.at[:, pl.ds(off2, 256)], ap, s2)
            c7.wait()
            c8.wait()

            @pl.loop(0, 16)
            def _(c):
                o16 = c * 16
                s = sp[0, pl.ds(o16, 16)]
                a = ap[0, pl.ds(o16, 16)]
                for r in range(1, 16):
                    s = s + sp[r, pl.ds(o16, 16)]
                    a = a + ap[r, pl.ds(o16, 16)]
                accs[pl.ds(o16, 16)] = s
                acca[pl.ds(o16, 16)] = a

            c9 = pltpu.async_copy(acca, anc_o.at[pl.ds(off2, 256)], s3)
            pltpu.sync_copy(cns_hbm.at[accs], rows)
            pltpu.sync_copy(rows, cst_o.at[pl.ds(off2, 256)])
            c9.wait()

    return body


def _final_body(cv_ref, anc_ref, pi_ref, nz_ref, out_o, row, sem):
    x = cv_ref[...]
    m = jnp.max(x)
    ii = (jax.lax.broadcasted_iota(jnp.int32, (NB, T), 0) * T
          + jax.lax.broadcasted_iota(jnp.int32, (NB, T), 1))
    flat = jnp.min(jnp.where(x == m, ii, jnp.int32(2 ** 30)))
    g = anc_ref[flat % T]
    cp = pltpu.make_async_copy(nz_ref.at[pl.ds(g, 1), :], row, sem)
    cp.start()
    cp.wait()
    out_o[...] = jnp.tanh(pi_ref[...] + row[...])


def _select(cv16, meta, offs, anc, cns, first):
    i32 = jnp.int32
    cst, anc_new = pl.kernel(
        _sc_select(first),
        out_type=[jax.ShapeDtypeStruct((T, SD), jnp.float32),
                  jax.ShapeDtypeStruct((T,), i32)],
        mesh=_sc_mesh(),
        compiler_params=_sc_params(),
        scratch_types=[pltpu.VMEM((4096,), i32),
                       pltpu.VMEM((16,), i32),
                       pltpu.VMEM((32,), i32),
                       pltpu.VMEM((T,), i32),
                       pltpu.VMEM((4160,), i32),
                       pltpu.VMEM((4160,), i32),
                       pltpu.VMEM((16, 256), i32),
                       pltpu.VMEM((16, 256), i32),
                       pltpu.VMEM((256,), i32),
                       pltpu.VMEM((256,), i32),
                       pltpu.VMEM((256, SD), jnp.float32),
                       pltpu.VMEM_SHARED((16, 4160), i32),
                       pltpu.VMEM_SHARED((16, 4160), i32),
                       pltpu.SemaphoreType.DMA,
                       pltpu.SemaphoreType.DMA,
                       pltpu.SemaphoreType.DMA,
                       pltpu.SemaphoreType.DMA],
    )(cv16, meta, offs.reshape(NB), anc, cns)
    return cst, anc_new


def kernel(s_t, adversary_strategy, W_m1, W_m2, W_a1, W_a2, W_p1, W_p2,
           W_h_a, W_h_s, W_v, w_health, noise):
    call = pl.pallas_call
    f32 = jnp.float32
    i32 = jnp.int32
    s2 = s_t.reshape(1, SD)
    adv2 = adversary_strategy.reshape(1, SD)
    whr = w_health.reshape(1, SD)

    vec = jax.ShapeDtypeStruct((1, SD), f32)
    rep = pl.BlockSpec((1, SD), lambda j: (0, 0))
    rep_c = pl.BlockSpec((SD, 1), lambda j: (0, 0))
    rep_m = pl.BlockSpec((SD, SD), lambda j: (0, 0))
    full16 = pl.BlockSpec((NB, T), lambda j: (0, 0))
    smem = pl.BlockSpec(memory_space=pltpu.SMEM)

    strat_a, strat_p, pi, strat, h0, cns, cv16, meta, offs = call(
        _bloom_body,
        grid=(NB,),
        in_specs=[rep, rep, rep_m, rep_m, rep_m, rep_m, rep_m, rep_m, rep_m,
                  rep, rep_m, rep_m,
                  pl.BlockSpec((T, SD), lambda j: (j, 0))],
        out_specs=[rep, rep, rep, rep, pl.BlockSpec((1, 1), lambda j: (0, 0)),
                   pl.BlockSpec((T, SD), lambda j: (j, 0)), full16, smem,
                   pl.BlockSpec((NB, 1), lambda j: (0, 0))],
        out_shape=[vec, vec, vec, vec, jax.ShapeDtypeStruct((1, 1), f32),
                   jax.ShapeDtypeStruct((N0, SD), f32),
                   jax.ShapeDtypeStruct((NB, T), i32),
                   jax.ShapeDtypeStruct((16,), i32),
                   jax.ShapeDtypeStruct((NB, 1), i32)],
        scratch_shapes=[pltpu.VMEM((NB, T), f32), pltpu.VMEM((4, SD), f32)],
    )(s2, adv2, W_m1, W_m2, W_a1, W_a2, W_p1, W_p2, W_h_s, whr, W_h_a, W_v,
      noise)

    noise16 = noise[:NB].reshape(NB, 1, SD)
    anc = jnp.zeros((T,), i32)
    for rnd in (1, 2):
        cst, anc = _select(cv16, meta, offs, anc, cns, rnd == 1)
        last = rnd == 2
        nz_spec = pl.BlockSpec((1, 1, SD),
                               lambda j: (jnp.maximum(j - 1, 0), 0, 0))
        outs = call(
            functools.partial(_round_body, last),
            grid=(NB + 1,),
            in_specs=[rep, rep, rep_m, rep_m, rep_m, rep, rep_m, rep_m,
                      rep, pl.BlockSpec((1, 1), lambda j: (0, 0)), nz_spec,
                      pl.BlockSpec((T, SD), lambda j: (0, 0))],
            out_specs=([full16] if last else
                       [pl.BlockSpec((T, SD),
                                     lambda j: (jnp.maximum(j - 1, 0), 0)),
                        full16, smem, pl.BlockSpec((NB, 1), lambda j: (0, 0))]),
            out_shape=([jax.ShapeDtypeStruct((NB, T), i32)] if last else
                       [jax.ShapeDtypeStruct((N0, SD), f32),
                        jax.ShapeDtypeStruct((NB, T), i32),
                        jax.ShapeDtypeStruct((16,), i32),
                        jax.ShapeDtypeStruct((NB, 1), i32)]),
            scratch_shapes=[pltpu.VMEM((NB, T), f32),
                            pltpu.VMEM((1, SD), f32),
                            pltpu.VMEM((T, SD), f32),
                            pltpu.VMEM((T, SD), f32)],
        )(strat_a, strat_p, W_a2, W_p2, W_h_s, whr, W_h_a, W_v, strat, h0,
          noise16, cst)
        if last:
            cv16 = outs if isinstance(outs, jax.Array) else outs[0]
        else:
            cns, cv16, meta, offs = outs

    out = call(
        _final_body,
        in_specs=[pl.BlockSpec((NB, T), lambda: (0, 0)), smem,
                  pl.BlockSpec((1, SD), lambda: (0, 0)),
                  pl.BlockSpec(memory_space=pl.ANY)],
        out_specs=pl.BlockSpec((1, SD), lambda: (0, 0)),
        out_shape=jax.ShapeDtypeStruct((1, SD), f32),
        scratch_shapes=[pltpu.VMEM((1, SD), f32), pltpu.SemaphoreType.DMA],
    )(cv16, anc, pi, noise)
    return out.reshape(SD)
